# baseline (speedup 1.0000x reference)
import jax
import jax.numpy as jnp
from jax.experimental import pallas as pl
from jax.experimental.pallas import tpu as pltpu

C, N, DIN, DHID, DOUT = 3, 4096, 128, 64, 32

BM_A = 512
BM_B = 512


def _pass_a_kernel(x_ref, w1_ref, b1_ref, w2_ref, adj_ref, s2_ref, s1_scr):
    i = pl.program_id(1)

    @pl.when(i == 0)
    def _():
        s1_scr[...] = jnp.dot(x_ref[0], w1_ref[0],
                              preferred_element_type=jnp.float32)

    h1 = jnp.dot(adj_ref[0], s1_scr[...],
                 preferred_element_type=jnp.float32)
    h1 = jnp.maximum(h1 + b1_ref[...], 0.0)
    s2_ref[0] = jnp.dot(h1, w2_ref[0], preferred_element_type=jnp.float32)


def _pass_b_kernel(adj_ref, s2_ref, w3_ref, bias_ref, out_ref):
    c = pl.program_id(1)
    acc = jnp.dot(adj_ref[0], s2_ref[0],
                  preferred_element_type=jnp.float32)
    for d in range(C):
        contrib = acc * w3_ref[d, c]

        @pl.when(c == 0)
        def _(contrib=contrib, d=d):
            out_ref[d] = contrib + bias_ref[d]

        @pl.when(c > 0)
        def _(contrib=contrib, d=d):
            out_ref[d] = out_ref[d] + contrib


def kernel(x, adj, W1, b1, W2, b2, W3, b3):
    b1r = b1.reshape(1, DHID)

    s2 = pl.pallas_call(
        _pass_a_kernel,
        grid=(C, N // BM_A),
        in_specs=[
            pl.BlockSpec((1, N, DIN), lambda c, i: (c, 0, 0)),
            pl.BlockSpec((1, DIN, DHID), lambda c, i: (c, 0, 0)),
            pl.BlockSpec((1, DHID), lambda c, i: (0, 0)),
            pl.BlockSpec((1, DHID, DOUT), lambda c, i: (c, 0, 0)),
            pl.BlockSpec((1, BM_A, N), lambda c, i: (c, i, 0)),
        ],
        out_specs=pl.BlockSpec((1, BM_A, DOUT), lambda c, i: (c, i, 0)),
        out_shape=jax.ShapeDtypeStruct((C, N, DOUT), jnp.float32),
        scratch_shapes=[pltpu.VMEM((N, DHID), jnp.float32)],
    )(x, W1, b1r, W2, adj)

    out_bias = (jnp.sum(W3, axis=1)[:, None] * b2[None, :]
                + b3[None, :])

    out = pl.pallas_call(
        _pass_b_kernel,
        grid=(N // BM_B, C),
        in_specs=[
            pl.BlockSpec((1, BM_B, N), lambda i, c: (c, i, 0)),
            pl.BlockSpec((1, N, DOUT), lambda i, c: (c, 0, 0)),
            pl.BlockSpec(memory_space=pltpu.SMEM),
            pl.BlockSpec((C, DOUT), lambda i, c: (0, 0)),
        ],
        out_specs=pl.BlockSpec((C, BM_B, DOUT), lambda i, c: (0, i, 0)),
        out_shape=jax.ShapeDtypeStruct((C, N, DOUT), jnp.float32),
    )(adj, s2, W3, out_bias)

    return out


# speedup vs baseline: 1.0576x; 1.0576x over previous
import jax
import jax.numpy as jnp
from jax.experimental import pallas as pl
from jax.experimental.pallas import tpu as pltpu

C, N, DIN, DHID, DOUT = 3, 4096, 128, 64, 32

BM_A = 512
BM_B = 512


def _pass_a_kernel(x_ref, w1_ref, b1_ref, w2_ref, adj_ref, s2_ref, s1_scr):
    i = pl.program_id(1)

    @pl.when(i == 0)
    def _():
        s1 = jnp.dot(x_ref[0], w1_ref[0],
                     preferred_element_type=jnp.float32)
        s1_scr[...] = s1.astype(jnp.bfloat16)

    h1 = jnp.dot(adj_ref[0].astype(jnp.bfloat16), s1_scr[...],
                 preferred_element_type=jnp.float32)
    h1 = jnp.maximum(h1 + b1_ref[...], 0.0)
    s2 = jnp.dot(h1, w2_ref[0], preferred_element_type=jnp.float32)
    s2_ref[0] = s2.astype(jnp.bfloat16)


def _pass_b_kernel(adj_ref, s2_ref, w3_ref, bias_ref, out_ref):
    c = pl.program_id(1)
    acc = jnp.dot(adj_ref[0].astype(jnp.bfloat16), s2_ref[0],
                  preferred_element_type=jnp.float32)
    for d in range(C):
        contrib = acc * w3_ref[d, c]

        @pl.when(c == 0)
        def _(contrib=contrib, d=d):
            out_ref[d] = contrib + bias_ref[d]

        @pl.when(c > 0)
        def _(contrib=contrib, d=d):
            out_ref[d] = out_ref[d] + contrib


def kernel(x, adj, W1, b1, W2, b2, W3, b3):
    b1r = b1.reshape(1, DHID)

    s2 = pl.pallas_call(
        _pass_a_kernel,
        grid=(C, N // BM_A),
        in_specs=[
            pl.BlockSpec((1, N, DIN), lambda c, i: (c, 0, 0)),
            pl.BlockSpec((1, DIN, DHID), lambda c, i: (c, 0, 0)),
            pl.BlockSpec((1, DHID), lambda c, i: (0, 0)),
            pl.BlockSpec((1, DHID, DOUT), lambda c, i: (c, 0, 0)),
            pl.BlockSpec((1, BM_A, N), lambda c, i: (c, i, 0)),
        ],
        out_specs=pl.BlockSpec((1, BM_A, DOUT), lambda c, i: (c, i, 0)),
        out_shape=jax.ShapeDtypeStruct((C, N, DOUT), jnp.bfloat16),
        scratch_shapes=[pltpu.VMEM((N, DHID), jnp.bfloat16)],
    )(x, W1, b1r, W2, adj)

    out_bias = (jnp.sum(W3, axis=1)[:, None] * b2[None, :]
                + b3[None, :])

    out = pl.pallas_call(
        _pass_b_kernel,
        grid=(N // BM_B, C),
        in_specs=[
            pl.BlockSpec((1, BM_B, N), lambda i, c: (c, i, 0)),
            pl.BlockSpec((1, N, DOUT), lambda i, c: (c, 0, 0)),
            pl.BlockSpec(memory_space=pltpu.SMEM),
            pl.BlockSpec((C, DOUT), lambda i, c: (0, 0)),
        ],
        out_specs=pl.BlockSpec((C, BM_B, DOUT), lambda i, c: (0, i, 0)),
        out_shape=jax.ShapeDtypeStruct((C, N, DOUT), jnp.float32),
    )(adj, s2, W3, out_bias)

    return out
